# baseline (speedup 1.0000x reference)
import functools

import jax
import jax.numpy as jnp
from jax import lax
from jax.experimental import pallas as pl
from jax.experimental.pallas import tpu as pltpu
from jax.experimental.pallas import tpu_sc as plsc

_GAMMA = 0.7
_NUM_MOLECULES = 20000

_NC = 2
_NS = 16
_CB = 4096
_CAP = 512


def _sc_invw_kernel(N, ids_hbm, invw_hbm, counts_sh, e0, e1, bi0, bv0, bi1,
                    bv1, ci0, cv0, ci1, cv1, tabv, aw0, aw1, si0, si1, ss0,
                    ss1, so0, so1):
    M = _NUM_MOLECULES
    cid = lax.axis_index("c")
    sid = lax.axis_index("s")

    zeros16 = jnp.zeros((16,), jnp.float32)

    for q in range(80):
        aw0[pl.ds(q * 16, 16)] = zeros16
    base_bins = sid * 1248

    @pl.when(sid < 15)
    def _():
        pltpu.sync_copy(aw0.at[pl.ds(0, 1248)],
                        counts_sh.at[pl.ds(base_bins, 1248)])

    @pl.when(sid == 15)
    def _():
        pltpu.sync_copy(aw0.at[pl.ds(0, 1280)],
                        counts_sh.at[pl.ds(base_bins, 1280)])

    sent = jnp.full((16,), -1, jnp.int32)
    e0[pl.ds(0, 16)] = sent
    e1[pl.ds(0, 16)] = sent
    e0[pl.ds(16 + _CB + 16, 16)] = sent
    e1[pl.ds(16 + _CB + 16, 16)] = sent
    plsc.subcore_barrier()

    tile_chunk = 62504
    tbase = sid * tile_chunk
    tend = jnp.minimum(tbase + tile_chunk, N)
    lanes = lax.iota(jnp.int32, 16)
    win = _CB + 16

    def win_start(c):
        return pl.multiple_of(
            jnp.clip(tbase + c * _CB - 8, 0, N - win), 8)

    def in_copy(c, ebuf, sem):
        return pltpu.make_async_copy(ids_hbm.at[pl.ds(win_start(c), win)],
                                     ebuf.at[pl.ds(16, win)], sem)

    zeros16 = jnp.zeros((16,), jnp.float32)
    izeros16 = jnp.zeros((16,), jnp.int32)

    def compute(c, ebuf, bi, bv, ci, cv, masked):
        base = tbase + c * _CB
        off = base - win_start(c)
        for z in range(_CAP // 16):
            ci[pl.ds(z * 16, 16)] = izeros16
            cv[pl.ds(z * 16, 16)] = zeros16

        def group_body(q, cursor):
            p = q * 16
            lo = 16 + off + p
            v = ebuf[pl.ds(lo, 16)]
            prev = ebuf[pl.ds(lo - 1, 16)]
            nxt = ebuf[pl.ds(lo + 1, 16)]
            g = base + p + lanes
            is_s = v != prev
            is_e = v != nxt
            val = (jnp.where(is_e, g + 1, 0) - jnp.where(is_s, g, 0))
            if masked:
                valid = g < tend
                idx = jnp.where(valid, v, 0)
                val = jnp.where(valid, val, 0)
            else:
                idx = v
            fval = val.astype(jnp.float32)
            bi[pl.ds(p, 16)] = idx
            bv[pl.ds(p, 16)] = fval
            m = val != 0
            gated = m & (cursor < _CAP - 16)
            at = jnp.minimum(cursor, _CAP - 16)
            plsc.store_compressed(ci.at[pl.ds(at, 16)], idx, mask=gated)
            plsc.store_compressed(cv.at[pl.ds(at, 16)], fval, mask=gated)
            return cursor + plsc.all_reduce_population_count(m)[0]

        cursor = lax.fori_loop(0, _CB // 16, group_body, jnp.int32(0),
                               unroll=8)

        @pl.when(cursor >= _CAP - 16)
        def _():
            for z in range(_CAP // 16):
                ci[pl.ds(z * 16, 16)] = izeros16
                cv[pl.ds(z * 16, 16)] = zeros16
            pltpu.sync_copy(bv, counts_sh.at[bi], add=True)

    def sc_issue(ci, cv, sem):
        pltpu.async_copy(cv, counts_sh.at[ci], sem, add=True)

    def sc_wait(ci, cv, sem):
        pltpu.make_async_copy(cv, counts_sh.at[ci], sem).wait()

    in_copy(0, e0, si0).start()

    def pair_body(j, carry):
        c0 = 2 * j
        in_copy(c0, e0, si0).wait()
        in_copy(c0 + 1, e1, si1).start()

        @pl.when(j > 0)
        def _():
            sc_wait(ci0, cv0, ss0)

        compute(c0, e0, bi0, bv0, ci0, cv0, False)
        sc_issue(ci0, cv0, ss0)

        in_copy(c0 + 1, e1, si1).wait()

        @pl.when(j < 7)
        def _():
            in_copy(c0 + 2, e0, si0).start()

        @pl.when(j > 0)
        def _():
            sc_wait(ci1, cv1, ss1)

        @pl.when(j < 7)
        def _():
            compute(c0 + 1, e1, bi1, bv1, ci1, cv1, False)

        @pl.when(j == 7)
        def _():
            compute(c0 + 1, e1, bi1, bv1, ci1, cv1, True)

        sc_issue(ci1, cv1, ss1)
        return carry

    with jax.named_scope("p1_scan"):
        lax.fori_loop(0, 8, pair_body, 0)
        sc_wait(ci0, cv0, ss0)
        sc_wait(ci1, cv1, ss1)
    with jax.named_scope("p1_barrier"):
        plsc.subcore_barrier()

    with jax.named_scope("p2_table"):
        pltpu.sync_copy(counts_sh, tabv)

        def p_body(i, acc):
            cnt = tabv[pl.ds(i * 16, 16)]
            return acc + jnp.where(cnt > 0, 1.0, 0.0).astype(jnp.float32)

        pacc = lax.fori_loop(0, M // 16, p_body,
                             jnp.zeros((16,), jnp.float32), unroll=4)
        p_total = plsc.cumsum(pacc)[15]

        def inv_body(i, carry):
            cnt = tabv[pl.ds(i * 16, 16)]
            inv = jnp.where(cnt > 0, 1.0 / (cnt * p_total), 0.0)
            tabv[pl.ds(i * 16, 16)] = inv
            return carry

        lax.fori_loop(0, M // 16, inv_body, 0, unroll=4)

    half = N // 2
    tile3 = 31256
    abase = pl.multiple_of(cid * half + sid * tile3, 8)

    def in3(c, ebuf, sem):
        return pltpu.make_async_copy(
            ids_hbm.at[pl.ds(abase + c * _CB, _CB)],
            ebuf.at[pl.ds(16, _CB)], sem)

    def out3(c, awb, sem):
        return pltpu.make_async_copy(
            awb, invw_hbm.at[pl.ds(abase + c * _CB, _CB)], sem)

    def gather(ebuf, awb, ngroups):
        def g_body(q, carry2):
            idxv = ebuf[pl.ds(16 + q * 16, 16)]
            awb[pl.ds(q * 16, 16)] = plsc.load_gather(tabv, [idxv])
            return carry2

        lax.fori_loop(0, ngroups, g_body, 0, unroll=4)

    with jax.named_scope("p3_gather"):
        ebufs, awbs, sins, souts = (e0, e1), (aw0, aw1), (si0, si1), (so0, so1)
        in3(0, e0, si0).start()
        for c in range(7):
            b = c % 2
            in3(c, ebufs[b], sins[b]).wait()
            if c + 1 < 7:
                in3(c + 1, ebufs[1 - b], sins[1 - b]).start()
            if c >= 2:
                out3(c - 2, awbs[b], souts[b]).wait()
            gather(ebufs[b], awbs[b], _CB // 16)
            out3(c, awbs[b], souts[b]).start()
        out3(5, aw1, so1).wait()
        out3(6, aw0, so0).wait()

        def tail(length):
            tb = pl.multiple_of(abase + 7 * _CB, 8)
            pltpu.sync_copy(ids_hbm.at[pl.ds(tb, length)],
                            e0.at[pl.ds(16, length)])
            gather(e0, aw0, length // 16)
            pltpu.sync_copy(aw0.at[pl.ds(0, length)],
                            invw_hbm.at[pl.ds(tb, length)])

        @pl.when(sid < 15)
        def _():
            tail(2584)

        @pl.when(sid == 15)
        def _():
            tail(2488)


def _sc_invw(molecule_id):
    N = molecule_id.shape[0]
    mesh = plsc.VectorSubcoreMesh(core_axis_name="c", subcore_axis_name="s")
    return pl.kernel(
        functools.partial(_sc_invw_kernel, N),
        out_type=jax.ShapeDtypeStruct((N,), jnp.float32),
        mesh=mesh,
        compiler_params=pltpu.CompilerParams(needs_layout_passes=False),
        scratch_types=[
            pltpu.VMEM_SHARED((_NUM_MOLECULES,), jnp.float32),
            pltpu.VMEM((8192,), jnp.int32),
            pltpu.VMEM((8192,), jnp.int32),
            pltpu.VMEM((_CB,), jnp.int32),
            pltpu.VMEM((_CB,), jnp.float32),
            pltpu.VMEM((_CB,), jnp.int32),
            pltpu.VMEM((_CB,), jnp.float32),
            pltpu.VMEM((_CAP,), jnp.int32),
            pltpu.VMEM((_CAP,), jnp.float32),
            pltpu.VMEM((_CAP,), jnp.int32),
            pltpu.VMEM((_CAP,), jnp.float32),
            pltpu.VMEM((_NUM_MOLECULES,), jnp.float32),
            pltpu.VMEM((_CB,), jnp.float32),
            pltpu.VMEM((_CB,), jnp.float32),
            pltpu.SemaphoreType.DMA,
            pltpu.SemaphoreType.DMA,
            pltpu.SemaphoreType.DMA,
            pltpu.SemaphoreType.DMA,
            pltpu.SemaphoreType.DMA,
            pltpu.SemaphoreType.DMA,
        ],
    )(molecule_id)


_BL = 8192


def _tc_loss_kernel(N, weights, s1, s2, s3, s4, tgt, iw, out, acc):
    i = pl.program_id(0)
    t = tgt[...]

    def sq(s):
        return jnp.square(s[...].reshape(3, _BL) - t)

    a = weights[0] * sq(s1)
    a += weights[1] * sq(s2)
    a += weights[2] * sq(s3)
    a += weights[3] * sq(s4)
    cols = i * _BL + lax.broadcasted_iota(jnp.int32, (1, _BL), 1)
    dw = jnp.sum(a, axis=0, keepdims=True)
    iwb = jnp.where(cols < N, iw[...].reshape(1, _BL), 0.0)

    @pl.when(i == 0)
    def _():
        acc[...] = jnp.zeros_like(acc)

    acc[...] += jnp.where(cols < N, dw, 0.0) * iwb

    @pl.when(i == pl.num_programs(0) - 1)
    def _():
        out[...] = jnp.sum(acc[...]).reshape(1, 1)


def _tc_loss(st, xt_t, invw, weights, N):
    grid = (N + _BL - 1) // _BL
    sspec = [pl.BlockSpec((1, 3, _BL), lambda i, k=k: (k, 0, i))
             for k in (1, 2, 3, 4)]
    return pl.pallas_call(
        functools.partial(_tc_loss_kernel, N, weights),
        grid=(grid,),
        in_specs=sspec + [pl.BlockSpec((3, _BL), lambda i: (0, i)),
                          pl.BlockSpec((_BL,), lambda i: (i,))],
        out_specs=pl.BlockSpec((1, 1), lambda i: (0, 0)),
        out_shape=jax.ShapeDtypeStruct((1, 1), jnp.float32),
        scratch_shapes=[pltpu.VMEM((1, _BL), jnp.float32)],
    )(st, st, st, st, xt_t, invw)


def kernel(states_x, x_target, molecule_id):
    N = molecule_id.shape[0]
    K = states_x.shape[0] - 1
    w = [_GAMMA ** (K - 1 - k) for k in range(K)]
    tot = sum(w)
    weights = tuple(float(x / tot) for x in w)

    invw = _sc_invw(molecule_id)

    st = jnp.transpose(states_x, (0, 2, 1))
    xt_t = jnp.transpose(x_target, (1, 0))
    loss = _tc_loss(st, xt_t, invw, weights, N)
    return loss[0, 0]


# speedup vs baseline: 1.0418x; 1.0418x over previous
import functools

import jax
import jax.numpy as jnp
from jax import lax
from jax.experimental import pallas as pl
from jax.experimental.pallas import tpu as pltpu
from jax.experimental.pallas import tpu_sc as plsc

_GAMMA = 0.7
_NUM_MOLECULES = 20000

_NC = 2
_NS = 16
_CB = 4096


def _sc_invw_kernel(N, ids_hbm, invw_hbm, counts_sh, e0, e1, bi0, bv0, bi1,
                    bv1, tabv, aw0, aw1, si0, si1, ss0, ss1, so0, so1):
    M = _NUM_MOLECULES
    cid = lax.axis_index("c")
    sid = lax.axis_index("s")

    zeros16 = jnp.zeros((16,), jnp.float32)

    for q in range(80):
        aw0[pl.ds(q * 16, 16)] = zeros16
    base_bins = sid * 1248

    @pl.when(sid < 15)
    def _():
        pltpu.sync_copy(aw0.at[pl.ds(0, 1248)],
                        counts_sh.at[pl.ds(base_bins, 1248)])

    @pl.when(sid == 15)
    def _():
        pltpu.sync_copy(aw0.at[pl.ds(0, 1280)],
                        counts_sh.at[pl.ds(base_bins, 1280)])

    sent = jnp.full((16,), -1, jnp.int32)
    e0[pl.ds(0, 16)] = sent
    e1[pl.ds(0, 16)] = sent
    e0[pl.ds(16 + _CB + 16, 16)] = sent
    e1[pl.ds(16 + _CB + 16, 16)] = sent
    plsc.subcore_barrier()

    tile_chunk = 62504
    tbase = sid * tile_chunk
    tend = jnp.minimum(tbase + tile_chunk, N)
    lanes = lax.iota(jnp.int32, 16)
    win = _CB + 16

    def win_start(c):
        return pl.multiple_of(
            jnp.clip(tbase + c * _CB - 8, 0, N - win), 8)

    def in_copy(c, ebuf, sem):
        return pltpu.make_async_copy(ids_hbm.at[pl.ds(win_start(c), win)],
                                     ebuf.at[pl.ds(16, win)], sem)

    def compute(c, ebuf, bi, bv, masked):
        base = tbase + c * _CB
        off = base - win_start(c)

        def group_body(q, carry2):
            p = q * 16
            lo = 16 + off + p
            v = ebuf[pl.ds(lo, 16)]
            prev = ebuf[pl.ds(lo - 1, 16)]
            nxt = ebuf[pl.ds(lo + 1, 16)]
            g = base + p + lanes
            is_s = v != prev
            is_e = v != nxt
            val = (jnp.where(is_e, g + 1, 0) - jnp.where(is_s, g, 0))
            if masked:
                valid = g < tend
                idx = jnp.where(valid, v, 0)
                val = jnp.where(valid, val, 0)
            else:
                idx = v
            bi[pl.ds(p, 16)] = idx
            bv[pl.ds(p, 16)] = val.astype(jnp.float32)
            return carry2

        lax.fori_loop(0, _CB // 16, group_body, 0, unroll=8)

    def sc_issue(bi, bv, sem):
        pltpu.async_copy(bv, counts_sh.at[bi], sem, add=True)

    def sc_wait(bi, bv, sem):
        pltpu.make_async_copy(bv, counts_sh.at[bi], sem).wait()

    in_copy(0, e0, si0).start()

    def pair_body(j, carry):
        c0 = 2 * j
        in_copy(c0, e0, si0).wait()
        in_copy(c0 + 1, e1, si1).start()

        @pl.when(j > 0)
        def _():
            sc_wait(bi0, bv0, ss0)

        compute(c0, e0, bi0, bv0, False)
        sc_issue(bi0, bv0, ss0)

        in_copy(c0 + 1, e1, si1).wait()

        @pl.when(j < 7)
        def _():
            in_copy(c0 + 2, e0, si0).start()

        @pl.when(j > 0)
        def _():
            sc_wait(bi1, bv1, ss1)

        @pl.when(j < 7)
        def _():
            compute(c0 + 1, e1, bi1, bv1, False)

        @pl.when(j == 7)
        def _():
            compute(c0 + 1, e1, bi1, bv1, True)

        sc_issue(bi1, bv1, ss1)
        return carry

    lax.fori_loop(0, 8, pair_body, 0)
    sc_wait(bi0, bv0, ss0)
    sc_wait(bi1, bv1, ss1)
    plsc.subcore_barrier()

    with jax.named_scope("p2_table"):
        pltpu.sync_copy(counts_sh, tabv)

        def p_body(i, acc):
            cnt = tabv[pl.ds(i * 16, 16)]
            return acc + jnp.where(cnt > 0, 1.0, 0.0).astype(jnp.float32)

        pacc = lax.fori_loop(0, M // 16, p_body,
                             jnp.zeros((16,), jnp.float32), unroll=4)
        p_total = plsc.cumsum(pacc)[15]

        def inv_body(i, carry):
            cnt = tabv[pl.ds(i * 16, 16)]
            inv = jnp.where(cnt > 0, 1.0 / (cnt * p_total), 0.0)
            tabv[pl.ds(i * 16, 16)] = inv
            return carry

        lax.fori_loop(0, M // 16, inv_body, 0, unroll=4)

    half = N // 2
    tile3 = 31256
    abase = pl.multiple_of(cid * half + sid * tile3, 8)

    def in3(c, ebuf, sem):
        return pltpu.make_async_copy(
            ids_hbm.at[pl.ds(abase + c * _CB, _CB)],
            ebuf.at[pl.ds(16, _CB)], sem)

    def out3(c, awb, sem):
        return pltpu.make_async_copy(
            awb, invw_hbm.at[pl.ds(abase + c * _CB, _CB)], sem)

    def gather(ebuf, awb, ngroups):
        def g_body(q, carry2):
            idxv = ebuf[pl.ds(16 + q * 16, 16)]
            awb[pl.ds(q * 16, 16)] = plsc.load_gather(tabv, [idxv])
            return carry2

        lax.fori_loop(0, ngroups, g_body, 0, unroll=4)

    with jax.named_scope("p3_gather"):
        ebufs, awbs, sins, souts = (e0, e1), (aw0, aw1), (si0, si1), (so0, so1)
        in3(0, e0, si0).start()
        for c in range(7):
            b = c % 2
            in3(c, ebufs[b], sins[b]).wait()
            if c + 1 < 7:
                in3(c + 1, ebufs[1 - b], sins[1 - b]).start()
            if c >= 2:
                out3(c - 2, awbs[b], souts[b]).wait()
            gather(ebufs[b], awbs[b], _CB // 16)
            out3(c, awbs[b], souts[b]).start()
        out3(5, aw1, so1).wait()
        out3(6, aw0, so0).wait()

        def tail(length):
            tb = pl.multiple_of(abase + 7 * _CB, 8)
            pltpu.sync_copy(ids_hbm.at[pl.ds(tb, length)],
                            e0.at[pl.ds(16, length)])
            gather(e0, aw0, length // 16)
            pltpu.sync_copy(aw0.at[pl.ds(0, length)],
                            invw_hbm.at[pl.ds(tb, length)])

        @pl.when(sid < 15)
        def _():
            tail(2584)

        @pl.when(sid == 15)
        def _():
            tail(2488)


def _sc_invw(molecule_id):
    N = molecule_id.shape[0]
    mesh = plsc.VectorSubcoreMesh(core_axis_name="c", subcore_axis_name="s")
    return pl.kernel(
        functools.partial(_sc_invw_kernel, N),
        out_type=jax.ShapeDtypeStruct((N,), jnp.float32),
        mesh=mesh,
        compiler_params=pltpu.CompilerParams(needs_layout_passes=False),
        scratch_types=[
            pltpu.VMEM_SHARED((_NUM_MOLECULES,), jnp.float32),
            pltpu.VMEM((8192,), jnp.int32),
            pltpu.VMEM((8192,), jnp.int32),
            pltpu.VMEM((_CB,), jnp.int32),
            pltpu.VMEM((_CB,), jnp.float32),
            pltpu.VMEM((_CB,), jnp.int32),
            pltpu.VMEM((_CB,), jnp.float32),
            pltpu.VMEM((_NUM_MOLECULES,), jnp.float32),
            pltpu.VMEM((_CB,), jnp.float32),
            pltpu.VMEM((_CB,), jnp.float32),
            pltpu.SemaphoreType.DMA,
            pltpu.SemaphoreType.DMA,
            pltpu.SemaphoreType.DMA,
            pltpu.SemaphoreType.DMA,
            pltpu.SemaphoreType.DMA,
            pltpu.SemaphoreType.DMA,
        ],
    )(molecule_id)


_BL = 8192


def _tc_loss_kernel(N, weights, s1, s2, s3, s4, tgt, iw, out, acc):
    i = pl.program_id(0)
    t = tgt[...]

    def sq(s):
        return jnp.square(s[...].reshape(3, _BL) - t)

    a = weights[0] * sq(s1)
    a += weights[1] * sq(s2)
    a += weights[2] * sq(s3)
    a += weights[3] * sq(s4)
    cols = i * _BL + lax.broadcasted_iota(jnp.int32, (1, _BL), 1)
    dw = jnp.sum(a, axis=0, keepdims=True)

    @pl.when(i == 0)
    def _():
        acc[...] = jnp.zeros_like(acc)

    acc[...] += jnp.where(cols < N, dw * iw[...].reshape(1, _BL), 0.0)

    @pl.when(i == pl.num_programs(0) - 1)
    def _():
        out[...] = jnp.sum(acc[...]).reshape(1, 1)


def _tc_loss(st, xt_t, invw, weights, N):
    grid = (N + _BL - 1) // _BL
    sspec = [pl.BlockSpec((1, 3, _BL), lambda i, k=k: (k, 0, i))
             for k in (1, 2, 3, 4)]
    return pl.pallas_call(
        functools.partial(_tc_loss_kernel, N, weights),
        grid=(grid,),
        in_specs=sspec + [pl.BlockSpec((3, _BL), lambda i: (0, i)),
                          pl.BlockSpec((_BL,), lambda i: (i,))],
        out_specs=pl.BlockSpec((1, 1), lambda i: (0, 0)),
        out_shape=jax.ShapeDtypeStruct((1, 1), jnp.float32),
        scratch_shapes=[pltpu.VMEM((1, _BL), jnp.float32)],
    )(st, st, st, st, xt_t, invw)


def kernel(states_x, x_target, molecule_id):
    N = molecule_id.shape[0]
    K = states_x.shape[0] - 1
    w = [_GAMMA ** (K - 1 - k) for k in range(K)]
    tot = sum(w)
    weights = tuple(float(x / tot) for x in w)

    invw = _sc_invw(molecule_id)

    st = jnp.transpose(states_x, (0, 2, 1))
    xt_t = jnp.transpose(x_target, (1, 0))
    loss = _tc_loss(st, xt_t, invw, weights, N)
    return loss[0, 0]


# speedup vs baseline: 1.0669x; 1.0241x over previous
import functools

import jax
import jax.numpy as jnp
from jax import lax
from jax.experimental import pallas as pl
from jax.experimental.pallas import tpu as pltpu
from jax.experimental.pallas import tpu_sc as plsc

_GAMMA = 0.7
_NUM_MOLECULES = 20000

_NC = 2
_NS = 16
_CB = 4096


def _sc_invw_kernel(N, ids_hbm, invw_hbm, counts_sh, e0, e1, bi0, bv0, bi1,
                    bv1, tabv, aw0, aw1, si0, si1, ss0, ss1, so0, so1):
    M = _NUM_MOLECULES
    cid = lax.axis_index("c")
    sid = lax.axis_index("s")

    zeros16 = jnp.zeros((16,), jnp.float32)

    for q in range(80):
        aw0[pl.ds(q * 16, 16)] = zeros16
    base_bins = sid * 1248

    @pl.when(sid < 15)
    def _():
        pltpu.sync_copy(aw0.at[pl.ds(0, 1248)],
                        counts_sh.at[pl.ds(base_bins, 1248)])

    @pl.when(sid == 15)
    def _():
        pltpu.sync_copy(aw0.at[pl.ds(0, 1280)],
                        counts_sh.at[pl.ds(base_bins, 1280)])

    sent = jnp.full((16,), -1, jnp.int32)
    e0[pl.ds(0, 16)] = sent
    e1[pl.ds(0, 16)] = sent
    e0[pl.ds(16 + _CB + 16, 16)] = sent
    e1[pl.ds(16 + _CB + 16, 16)] = sent
    plsc.subcore_barrier()

    tile_chunk = 62504
    tbase = sid * tile_chunk
    tend = jnp.minimum(tbase + tile_chunk, N)
    lanes = lax.iota(jnp.int32, 16)
    win = _CB + 16

    def win_start(c):
        return pl.multiple_of(
            jnp.clip(tbase + c * _CB - 8, 0, N - win), 8)

    def in_copy(c, ebuf, sem):
        return pltpu.make_async_copy(ids_hbm.at[pl.ds(win_start(c), win)],
                                     ebuf.at[pl.ds(16, win)], sem)

    def compute(c, ebuf, bi, bv, masked):
        base = tbase + c * _CB
        off = base - win_start(c)

        def group_body(q, carry2):
            p = q * 16
            lo = 16 + off + p
            v = ebuf[pl.ds(lo, 16)]
            prev = ebuf[pl.ds(lo - 1, 16)]
            nxt = ebuf[pl.ds(lo + 1, 16)]
            g = base + p + lanes
            is_s = v != prev
            is_e = v != nxt
            val = (jnp.where(is_e, g + 1, 0) - jnp.where(is_s, g, 0))
            if masked:
                valid = g < tend
                idx = jnp.where(valid, v, 0)
                val = jnp.where(valid, val, 0)
            else:
                idx = v
            bi[pl.ds(p, 16)] = idx
            bv[pl.ds(p, 16)] = val.astype(jnp.float32)
            return carry2

        lax.fori_loop(0, _CB // 16, group_body, 0, unroll=8)

    def sc_issue(bi, bv, sem):
        pltpu.async_copy(bv, counts_sh.at[bi], sem, add=True)

    def sc_wait(bi, bv, sem):
        pltpu.make_async_copy(bv, counts_sh.at[bi], sem).wait()

    in_copy(0, e0, si0).start()

    def pair_body(j, carry):
        c0 = 2 * j
        in_copy(c0, e0, si0).wait()
        in_copy(c0 + 1, e1, si1).start()

        @pl.when(j > 0)
        def _():
            sc_wait(bi0, bv0, ss0)

        compute(c0, e0, bi0, bv0, False)
        sc_issue(bi0, bv0, ss0)

        in_copy(c0 + 1, e1, si1).wait()

        @pl.when(j < 7)
        def _():
            in_copy(c0 + 2, e0, si0).start()

        @pl.when(j > 0)
        def _():
            sc_wait(bi1, bv1, ss1)

        @pl.when(j < 7)
        def _():
            compute(c0 + 1, e1, bi1, bv1, False)

        @pl.when(j == 7)
        def _():
            compute(c0 + 1, e1, bi1, bv1, True)

        sc_issue(bi1, bv1, ss1)
        return carry

    lax.fori_loop(0, 8, pair_body, 0)
    sc_wait(bi0, bv0, ss0)
    sc_wait(bi1, bv1, ss1)
    plsc.subcore_barrier()

    with jax.named_scope("p2_table"):
        pltpu.sync_copy(counts_sh, tabv)

        def p_body(i, acc):
            cnt = tabv[pl.ds(i * 16, 16)]
            return acc + jnp.where(cnt > 0, 1.0, 0.0).astype(jnp.float32)

        pacc = lax.fori_loop(0, M // 16, p_body,
                             jnp.zeros((16,), jnp.float32), unroll=4)
        p_total = plsc.cumsum(pacc)[15]

        def inv_body(i, carry):
            cnt = tabv[pl.ds(i * 16, 16)]
            inv = jnp.where(cnt > 0, 1.0 / (cnt * p_total), 0.0)
            tabv[pl.ds(i * 16, 16)] = inv
            return carry

        lax.fori_loop(0, M // 16, inv_body, 0, unroll=4)

    half = N // 2
    tile3 = 31256
    abase = pl.multiple_of(cid * half + sid * tile3, 8)

    def in3(c, ebuf, sem):
        return pltpu.make_async_copy(
            ids_hbm.at[pl.ds(abase + c * _CB, _CB)],
            ebuf.at[pl.ds(16, _CB)], sem)

    def out3(c, awb, sem):
        return pltpu.make_async_copy(
            awb, invw_hbm.at[pl.ds(abase + c * _CB, _CB)], sem)

    def gather(ebuf, awb, ngroups):
        def g_body(q, carry2):
            idxv = ebuf[pl.ds(16 + q * 16, 16)]
            awb[pl.ds(q * 16, 16)] = plsc.load_gather(tabv, [idxv])
            return carry2

        lax.fori_loop(0, ngroups, g_body, 0, unroll=4)

    with jax.named_scope("p3_gather"):
        ebufs, awbs, sins, souts = (e0, e1), (aw0, aw1), (si0, si1), (so0, so1)
        in3(0, e0, si0).start()
        for c in range(7):
            b = c % 2
            in3(c, ebufs[b], sins[b]).wait()
            if c + 1 < 7:
                in3(c + 1, ebufs[1 - b], sins[1 - b]).start()
            if c >= 2:
                out3(c - 2, awbs[b], souts[b]).wait()
            gather(ebufs[b], awbs[b], _CB // 16)
            out3(c, awbs[b], souts[b]).start()
        out3(5, aw1, so1).wait()
        out3(6, aw0, so0).wait()

        def tail(length):
            tb = pl.multiple_of(abase + 7 * _CB, 8)
            pltpu.sync_copy(ids_hbm.at[pl.ds(tb, length)],
                            e0.at[pl.ds(16, length)])
            gather(e0, aw0, length // 16)
            pltpu.sync_copy(aw0.at[pl.ds(0, length)],
                            invw_hbm.at[pl.ds(tb, length)])

        @pl.when(sid < 15)
        def _():
            tail(2584)

        @pl.when(sid == 15)
        def _():
            tail(2488)


def _sc_invw(molecule_id):
    N = molecule_id.shape[0]
    mesh = plsc.VectorSubcoreMesh(core_axis_name="c", subcore_axis_name="s")
    return pl.kernel(
        functools.partial(_sc_invw_kernel, N),
        out_type=jax.ShapeDtypeStruct((N,), jnp.float32),
        mesh=mesh,
        compiler_params=pltpu.CompilerParams(needs_layout_passes=False),
        scratch_types=[
            pltpu.VMEM_SHARED((_NUM_MOLECULES,), jnp.float32),
            pltpu.VMEM((8192,), jnp.int32),
            pltpu.VMEM((8192,), jnp.int32),
            pltpu.VMEM((_CB,), jnp.int32),
            pltpu.VMEM((_CB,), jnp.float32),
            pltpu.VMEM((_CB,), jnp.int32),
            pltpu.VMEM((_CB,), jnp.float32),
            pltpu.VMEM((_NUM_MOLECULES,), jnp.float32),
            pltpu.VMEM((_CB,), jnp.float32),
            pltpu.VMEM((_CB,), jnp.float32),
            pltpu.SemaphoreType.DMA,
            pltpu.SemaphoreType.DMA,
            pltpu.SemaphoreType.DMA,
            pltpu.SemaphoreType.DMA,
            pltpu.SemaphoreType.DMA,
            pltpu.SemaphoreType.DMA,
        ],
    )(molecule_id)


_BL = 8192


def _tc_loss_kernel(N, weights, s1, s2, s3, s4, tgt, iw, out, acc):
    i = pl.program_id(0)
    t = tgt[...]

    def sq(s):
        return jnp.square(s[...].reshape(3, _BL) - t)

    a = weights[0] * sq(s1)
    a += weights[1] * sq(s2)
    a += weights[2] * sq(s3)
    a += weights[3] * sq(s4)
    prod = a * iw[...].reshape(1, _BL)

    @pl.when(i == 0)
    def _():
        acc[...] = jnp.zeros_like(acc)

    last = pl.num_programs(0) - 1

    @pl.when(i < last)
    def _():
        acc[...] += prod

    @pl.when(i == last)
    def _():
        cols = i * _BL + lax.broadcasted_iota(jnp.int32, (1, _BL), 1)
        acc[...] += jnp.where(cols < N, prod, 0.0)
        out[...] = jnp.sum(acc[...]).reshape(1, 1)


def _tc_loss(st, xt_t, invw, weights, N):
    grid = (N + _BL - 1) // _BL
    sspec = [pl.BlockSpec((1, 3, _BL), lambda i, k=k: (k, 0, i))
             for k in (1, 2, 3, 4)]
    return pl.pallas_call(
        functools.partial(_tc_loss_kernel, N, weights),
        grid=(grid,),
        in_specs=sspec + [pl.BlockSpec((3, _BL), lambda i: (0, i)),
                          pl.BlockSpec((_BL,), lambda i: (i,))],
        out_specs=pl.BlockSpec((1, 1), lambda i: (0, 0)),
        out_shape=jax.ShapeDtypeStruct((1, 1), jnp.float32),
        scratch_shapes=[pltpu.VMEM((3, _BL), jnp.float32)],
    )(st, st, st, st, xt_t, invw)


def kernel(states_x, x_target, molecule_id):
    N = molecule_id.shape[0]
    K = states_x.shape[0] - 1
    w = [_GAMMA ** (K - 1 - k) for k in range(K)]
    tot = sum(w)
    weights = tuple(float(x / tot) for x in w)

    invw = _sc_invw(molecule_id)

    st = jnp.transpose(states_x, (0, 2, 1))
    xt_t = jnp.transpose(x_target, (1, 0))
    loss = _tc_loss(st, xt_t, invw, weights, N)
    return loss[0, 0]


# speedup vs baseline: 1.2037x; 1.1283x over previous
import functools

import jax
import jax.numpy as jnp
from jax import lax
from jax.experimental import pallas as pl
from jax.experimental.pallas import tpu as pltpu
from jax.experimental.pallas import tpu_sc as plsc

_GAMMA = 0.7
_NUM_MOLECULES = 20000

_NC = 2
_NS = 16
_CB = 4096


def _sc_invw_kernel(N, ids_hbm, invw_hbm, counts_sh, e0, e1, bi0, bv0, bi1,
                    bv1, tabv, aw0, aw1, si0, si1, ss0, ss1, so0, so1):
    M = _NUM_MOLECULES
    cid = lax.axis_index("c")
    sid = lax.axis_index("s")

    zeros16 = jnp.zeros((16,), jnp.float32)

    for q in range(80):
        aw0[pl.ds(q * 16, 16)] = zeros16
    base_bins = sid * 1248

    @pl.when(sid < 15)
    def _():
        pltpu.sync_copy(aw0.at[pl.ds(0, 1248)],
                        counts_sh.at[pl.ds(base_bins, 1248)])

    @pl.when(sid == 15)
    def _():
        pltpu.sync_copy(aw0.at[pl.ds(0, 1280)],
                        counts_sh.at[pl.ds(base_bins, 1280)])

    sent = jnp.full((16,), -1, jnp.int32)
    e0[pl.ds(0, 16)] = sent
    e1[pl.ds(0, 16)] = sent
    e0[pl.ds(16 + _CB + 16, 16)] = sent
    e1[pl.ds(16 + _CB + 16, 16)] = sent
    plsc.subcore_barrier()

    tile_chunk = 62504
    tbase = sid * tile_chunk
    tend = jnp.minimum(tbase + tile_chunk, N)
    lanes = lax.iota(jnp.int32, 16)
    win = _CB + 16

    def win_start(c):
        return pl.multiple_of(
            jnp.clip(tbase + c * _CB - 8, 0, N - win), 8)

    def in_copy(c, ebuf, sem):
        return pltpu.make_async_copy(ids_hbm.at[pl.ds(win_start(c), win)],
                                     ebuf.at[pl.ds(16, win)], sem)

    def compute(c, ebuf, bi, bv, masked):
        base = tbase + c * _CB
        off = base - win_start(c)

        def group_body(q, carry2):
            p = q * 16
            lo = 16 + off + p
            v = ebuf[pl.ds(lo, 16)]
            prev = ebuf[pl.ds(lo - 1, 16)]
            nxt = ebuf[pl.ds(lo + 1, 16)]
            g = base + p + lanes
            is_s = v != prev
            is_e = v != nxt
            val = (jnp.where(is_e, g + 1, 0) - jnp.where(is_s, g, 0))
            if masked:
                valid = g < tend
                idx = jnp.where(valid, v, 0)
                val = jnp.where(valid, val, 0)
            else:
                idx = v
            bi[pl.ds(p, 16)] = idx
            bv[pl.ds(p, 16)] = val.astype(jnp.float32)
            return carry2

        lax.fori_loop(0, _CB // 16, group_body, 0, unroll=8)

    def sc_issue(bi, bv, sem):
        pltpu.async_copy(bv, counts_sh.at[bi], sem, add=True)

    def sc_wait(bi, bv, sem):
        pltpu.make_async_copy(bv, counts_sh.at[bi], sem).wait()

    in_copy(0, e0, si0).start()

    def pair_body(j, carry):
        c0 = 2 * j
        in_copy(c0, e0, si0).wait()
        in_copy(c0 + 1, e1, si1).start()

        @pl.when(j > 0)
        def _():
            sc_wait(bi0, bv0, ss0)

        compute(c0, e0, bi0, bv0, False)
        sc_issue(bi0, bv0, ss0)

        in_copy(c0 + 1, e1, si1).wait()

        @pl.when(j < 7)
        def _():
            in_copy(c0 + 2, e0, si0).start()

        @pl.when(j > 0)
        def _():
            sc_wait(bi1, bv1, ss1)

        @pl.when(j < 7)
        def _():
            compute(c0 + 1, e1, bi1, bv1, False)

        @pl.when(j == 7)
        def _():
            compute(c0 + 1, e1, bi1, bv1, True)

        sc_issue(bi1, bv1, ss1)
        return carry

    lax.fori_loop(0, 8, pair_body, 0)
    sc_wait(bi0, bv0, ss0)
    sc_wait(bi1, bv1, ss1)
    plsc.subcore_barrier()

    with jax.named_scope("p2_table"):
        pltpu.sync_copy(counts_sh, tabv)

        def p_body(i, acc):
            cnt = tabv[pl.ds(i * 16, 16)]
            return acc + jnp.where(cnt > 0, 1.0, 0.0).astype(jnp.float32)

        pacc = lax.fori_loop(0, M // 16, p_body,
                             jnp.zeros((16,), jnp.float32), unroll=4)
        p_total = plsc.cumsum(pacc)[15]

        def inv_body(i, carry):
            cnt = tabv[pl.ds(i * 16, 16)]
            inv = jnp.where(cnt > 0, 1.0 / (cnt * p_total), 0.0)
            tabv[pl.ds(i * 16, 16)] = inv
            return carry

        lax.fori_loop(0, M // 16, inv_body, 0, unroll=4)

    half = N // 2
    tile3 = 31256
    abase = pl.multiple_of(cid * half + sid * tile3, 8)

    def in3(c, ebuf, sem):
        return pltpu.make_async_copy(
            ids_hbm.at[pl.ds(abase + c * _CB, _CB)],
            ebuf.at[pl.ds(16, _CB)], sem)

    def out3(c, awb, sem):
        return pltpu.make_async_copy(
            awb, invw_hbm.at[pl.ds(abase + c * _CB, _CB)], sem)

    def gather(ebuf, awb, ngroups):
        def g_body(q, carry2):
            idxv = ebuf[pl.ds(16 + q * 16, 16)]
            awb[pl.ds(q * 16, 16)] = plsc.load_gather(tabv, [idxv])
            return carry2

        lax.fori_loop(0, ngroups, g_body, 0, unroll=4)

    with jax.named_scope("p3_gather"):
        ebufs, awbs, sins, souts = (e0, e1), (aw0, aw1), (si0, si1), (so0, so1)
        in3(0, e0, si0).start()
        for c in range(7):
            b = c % 2
            in3(c, ebufs[b], sins[b]).wait()
            if c + 1 < 7:
                in3(c + 1, ebufs[1 - b], sins[1 - b]).start()
            if c >= 2:
                out3(c - 2, awbs[b], souts[b]).wait()
            gather(ebufs[b], awbs[b], _CB // 16)
            out3(c, awbs[b], souts[b]).start()
        out3(5, aw1, so1).wait()
        out3(6, aw0, so0).wait()

        def tail(length):
            tb = pl.multiple_of(abase + 7 * _CB, 8)
            pltpu.sync_copy(ids_hbm.at[pl.ds(tb, length)],
                            e0.at[pl.ds(16, length)])
            gather(e0, aw0, length // 16)
            pltpu.sync_copy(aw0.at[pl.ds(0, length)],
                            invw_hbm.at[pl.ds(tb, length)])

        @pl.when(sid < 15)
        def _():
            tail(2584)

        @pl.when(sid == 15)
        def _():
            tail(2488)


def _sc_invw(molecule_id):
    N = molecule_id.shape[0]
    mesh = plsc.VectorSubcoreMesh(core_axis_name="c", subcore_axis_name="s")
    return pl.kernel(
        functools.partial(_sc_invw_kernel, N),
        out_type=jax.ShapeDtypeStruct((N,), jnp.float32),
        mesh=mesh,
        compiler_params=pltpu.CompilerParams(needs_layout_passes=False),
        scratch_types=[
            pltpu.VMEM_SHARED((_NUM_MOLECULES,), jnp.float32),
            pltpu.VMEM((8192,), jnp.int32),
            pltpu.VMEM((8192,), jnp.int32),
            pltpu.VMEM((_CB,), jnp.int32),
            pltpu.VMEM((_CB,), jnp.float32),
            pltpu.VMEM((_CB,), jnp.int32),
            pltpu.VMEM((_CB,), jnp.float32),
            pltpu.VMEM((_NUM_MOLECULES,), jnp.float32),
            pltpu.VMEM((_CB,), jnp.float32),
            pltpu.VMEM((_CB,), jnp.float32),
            pltpu.SemaphoreType.DMA,
            pltpu.SemaphoreType.DMA,
            pltpu.SemaphoreType.DMA,
            pltpu.SemaphoreType.DMA,
            pltpu.SemaphoreType.DMA,
            pltpu.SemaphoreType.DMA,
        ],
    )(molecule_id)


_BL = 16384


def _tc_loss_kernel(N, weights, s1, s2, s3, s4, tgt, iw, out, acc):
    i = pl.program_id(0)
    t = tgt[...]

    def sq(s):
        return jnp.square(s[...].reshape(3, _BL) - t)

    a = weights[0] * sq(s1)
    a += weights[1] * sq(s2)
    a += weights[2] * sq(s3)
    a += weights[3] * sq(s4)
    prod = a * iw[...].reshape(1, _BL)

    @pl.when(i == 0)
    def _():
        acc[...] = jnp.zeros_like(acc)

    last = pl.num_programs(0) - 1

    @pl.when(i < last)
    def _():
        acc[...] += prod

    @pl.when(i == last)
    def _():
        cols = i * _BL + lax.broadcasted_iota(jnp.int32, (1, _BL), 1)
        acc[...] += jnp.where(cols < N, prod, 0.0)
        out[...] = jnp.sum(acc[...]).reshape(1, 1)


def _tc_loss(st, xt_t, invw, weights, N):
    grid = (N + _BL - 1) // _BL
    sspec = [pl.BlockSpec((1, 3, _BL), lambda i, k=k: (k, 0, i))
             for k in (1, 2, 3, 4)]
    return pl.pallas_call(
        functools.partial(_tc_loss_kernel, N, weights),
        grid=(grid,),
        in_specs=sspec + [pl.BlockSpec((3, _BL), lambda i: (0, i)),
                          pl.BlockSpec((_BL,), lambda i: (i,))],
        out_specs=pl.BlockSpec((1, 1), lambda i: (0, 0)),
        out_shape=jax.ShapeDtypeStruct((1, 1), jnp.float32),
        scratch_shapes=[pltpu.VMEM((3, _BL), jnp.float32)],
    )(st, st, st, st, xt_t, invw)


def kernel(states_x, x_target, molecule_id):
    N = molecule_id.shape[0]
    K = states_x.shape[0] - 1
    w = [_GAMMA ** (K - 1 - k) for k in range(K)]
    tot = sum(w)
    weights = tuple(float(x / tot) for x in w)

    invw = _sc_invw(molecule_id)

    st = jnp.transpose(states_x, (0, 2, 1))
    xt_t = jnp.transpose(x_target, (1, 0))
    loss = _tc_loss(st, xt_t, invw, weights, N)
    return loss[0, 0]


# speedup vs baseline: 1.2951x; 1.0759x over previous
import functools

import jax
import jax.numpy as jnp
from jax import lax
from jax.experimental import pallas as pl
from jax.experimental.pallas import tpu as pltpu
from jax.experimental.pallas import tpu_sc as plsc

_GAMMA = 0.7
_NUM_MOLECULES = 20000

_NC = 2
_NS = 16
_CB = 4096


def _sc_invw_kernel(N, ids_hbm, invw_hbm, counts_sh, e0, e1, bi0, bv0, bi1,
                    bv1, tabv, aw0, aw1, si0, si1, ss0, ss1, so0, so1):
    M = _NUM_MOLECULES
    cid = lax.axis_index("c")
    sid = lax.axis_index("s")

    zeros16 = jnp.zeros((16,), jnp.float32)

    for q in range(80):
        aw0[pl.ds(q * 16, 16)] = zeros16
    base_bins = sid * 1248

    @pl.when(sid < 15)
    def _():
        pltpu.sync_copy(aw0.at[pl.ds(0, 1248)],
                        counts_sh.at[pl.ds(base_bins, 1248)])

    @pl.when(sid == 15)
    def _():
        pltpu.sync_copy(aw0.at[pl.ds(0, 1280)],
                        counts_sh.at[pl.ds(base_bins, 1280)])

    sent = jnp.full((16,), -1, jnp.int32)
    e0[pl.ds(0, 16)] = sent
    e1[pl.ds(0, 16)] = sent
    e0[pl.ds(16 + _CB + 16, 16)] = sent
    e1[pl.ds(16 + _CB + 16, 16)] = sent
    plsc.subcore_barrier()

    tile_chunk = 62504
    tbase = sid * tile_chunk
    tend = jnp.minimum(tbase + tile_chunk, N)
    lanes = lax.iota(jnp.int32, 16)
    win = _CB + 16

    def win_start(c):
        return pl.multiple_of(
            jnp.clip(tbase + c * _CB - 8, 0, N - win), 8)

    def in_copy(c, ebuf, sem):
        return pltpu.make_async_copy(ids_hbm.at[pl.ds(win_start(c), win)],
                                     ebuf.at[pl.ds(16, win)], sem)

    def compute(c, ebuf, bi, bv, masked):
        base = tbase + c * _CB
        off = base - win_start(c)

        def group_body(q, carry2):
            p = q * 16
            lo = 16 + off + p
            v = ebuf[pl.ds(lo, 16)]
            prev = ebuf[pl.ds(lo - 1, 16)]
            nxt = ebuf[pl.ds(lo + 1, 16)]
            g = base + p + lanes
            is_s = v != prev
            is_e = v != nxt
            val = (jnp.where(is_e, g + 1, 0) - jnp.where(is_s, g, 0))
            if masked:
                valid = g < tend
                idx = jnp.where(valid, v, 0)
                val = jnp.where(valid, val, 0)
            else:
                idx = v
            bi[pl.ds(p, 16)] = idx
            bv[pl.ds(p, 16)] = val.astype(jnp.float32)
            return carry2

        lax.fori_loop(0, _CB // 16, group_body, 0, unroll=8)

    def sc_issue(bi, bv, sem):
        pltpu.async_copy(bv, counts_sh.at[bi], sem, add=True)

    def sc_wait(bi, bv, sem):
        pltpu.make_async_copy(bv, counts_sh.at[bi], sem).wait()

    in_copy(0, e0, si0).start()

    def pair_body(j, carry):
        c0 = 2 * j
        in_copy(c0, e0, si0).wait()
        in_copy(c0 + 1, e1, si1).start()

        @pl.when(j > 0)
        def _():
            sc_wait(bi0, bv0, ss0)

        compute(c0, e0, bi0, bv0, False)
        sc_issue(bi0, bv0, ss0)

        in_copy(c0 + 1, e1, si1).wait()

        @pl.when(j < 7)
        def _():
            in_copy(c0 + 2, e0, si0).start()

        @pl.when(j > 0)
        def _():
            sc_wait(bi1, bv1, ss1)

        @pl.when(j < 7)
        def _():
            compute(c0 + 1, e1, bi1, bv1, False)

        @pl.when(j == 7)
        def _():
            compute(c0 + 1, e1, bi1, bv1, True)

        sc_issue(bi1, bv1, ss1)
        return carry

    lax.fori_loop(0, 8, pair_body, 0)
    sc_wait(bi0, bv0, ss0)
    sc_wait(bi1, bv1, ss1)
    plsc.subcore_barrier()

    with jax.named_scope("p2_table"):
        pltpu.sync_copy(counts_sh, tabv)

        def p_body(i, acc):
            cnt = tabv[pl.ds(i * 16, 16)]
            return acc + jnp.where(cnt > 0, 1.0, 0.0).astype(jnp.float32)

        pacc = lax.fori_loop(0, M // 16, p_body,
                             jnp.zeros((16,), jnp.float32), unroll=4)
        p_total = plsc.cumsum(pacc)[15]

        def inv_body(i, carry):
            cnt = tabv[pl.ds(i * 16, 16)]
            inv = jnp.where(cnt > 0, 1.0 / (cnt * p_total), 0.0)
            tabv[pl.ds(i * 16, 16)] = inv
            return carry

        lax.fori_loop(0, M // 16, inv_body, 0, unroll=4)

    half = N // 2
    tile3 = 31256
    abase = pl.multiple_of(cid * half + sid * tile3, 8)

    def in3(c, ebuf, sem):
        return pltpu.make_async_copy(
            ids_hbm.at[pl.ds(abase + c * _CB, _CB)],
            ebuf.at[pl.ds(16, _CB)], sem)

    def out3(c, awb, sem):
        return pltpu.make_async_copy(
            awb, invw_hbm.at[pl.ds(abase + c * _CB, _CB)], sem)

    def gather(ebuf, awb, ngroups):
        def g_body(q, carry2):
            idxv = ebuf[pl.ds(16 + q * 16, 16)]
            awb[pl.ds(q * 16, 16)] = plsc.load_gather(tabv, [idxv])
            return carry2

        lax.fori_loop(0, ngroups, g_body, 0, unroll=4)

    with jax.named_scope("p3_gather"):
        ebufs, awbs, sins, souts = (e0, e1), (aw0, aw1), (si0, si1), (so0, so1)
        in3(0, e0, si0).start()
        for c in range(7):
            b = c % 2
            in3(c, ebufs[b], sins[b]).wait()
            if c + 1 < 7:
                in3(c + 1, ebufs[1 - b], sins[1 - b]).start()
            if c >= 2:
                out3(c - 2, awbs[b], souts[b]).wait()
            gather(ebufs[b], awbs[b], _CB // 16)
            out3(c, awbs[b], souts[b]).start()
        out3(5, aw1, so1).wait()
        out3(6, aw0, so0).wait()

        def tail(length):
            tb = pl.multiple_of(abase + 7 * _CB, 8)
            pltpu.sync_copy(ids_hbm.at[pl.ds(tb, length)],
                            e0.at[pl.ds(16, length)])
            gather(e0, aw0, length // 16)
            pltpu.sync_copy(aw0.at[pl.ds(0, length)],
                            invw_hbm.at[pl.ds(tb, length)])

        @pl.when(sid < 15)
        def _():
            tail(2584)

        @pl.when(sid == 15)
        def _():
            tail(2488)


def _sc_invw(molecule_id):
    N = molecule_id.shape[0]
    mesh = plsc.VectorSubcoreMesh(core_axis_name="c", subcore_axis_name="s")
    return pl.kernel(
        functools.partial(_sc_invw_kernel, N),
        out_type=jax.ShapeDtypeStruct((N,), jnp.float32),
        mesh=mesh,
        compiler_params=pltpu.CompilerParams(needs_layout_passes=False),
        scratch_types=[
            pltpu.VMEM_SHARED((_NUM_MOLECULES,), jnp.float32),
            pltpu.VMEM((8192,), jnp.int32),
            pltpu.VMEM((8192,), jnp.int32),
            pltpu.VMEM((_CB,), jnp.int32),
            pltpu.VMEM((_CB,), jnp.float32),
            pltpu.VMEM((_CB,), jnp.int32),
            pltpu.VMEM((_CB,), jnp.float32),
            pltpu.VMEM((_NUM_MOLECULES,), jnp.float32),
            pltpu.VMEM((_CB,), jnp.float32),
            pltpu.VMEM((_CB,), jnp.float32),
            pltpu.SemaphoreType.DMA,
            pltpu.SemaphoreType.DMA,
            pltpu.SemaphoreType.DMA,
            pltpu.SemaphoreType.DMA,
            pltpu.SemaphoreType.DMA,
            pltpu.SemaphoreType.DMA,
        ],
    )(molecule_id)


_BL = 32768


def _tc_loss_kernel(N, weights, s1, s2, s3, s4, tgt, iw, out, acc):
    i = pl.program_id(0)
    t = tgt[...]

    def sq(s):
        return jnp.square(s[...].reshape(3, _BL) - t)

    a = weights[0] * sq(s1)
    a += weights[1] * sq(s2)
    a += weights[2] * sq(s3)
    a += weights[3] * sq(s4)
    prod = a * iw[...].reshape(1, _BL)

    @pl.when(i == 0)
    def _():
        acc[...] = jnp.zeros_like(acc)

    last = pl.num_programs(0) - 1

    @pl.when(i < last)
    def _():
        acc[...] += prod

    @pl.when(i == last)
    def _():
        cols = i * _BL + lax.broadcasted_iota(jnp.int32, (1, _BL), 1)
        acc[...] += jnp.where(cols < N, prod, 0.0)
        out[...] = jnp.sum(acc[...]).reshape(1, 1)


def _tc_loss(st, xt_t, invw, weights, N):
    grid = (N + _BL - 1) // _BL
    sspec = [pl.BlockSpec((1, 3, _BL), lambda i, k=k: (k, 0, i))
             for k in (1, 2, 3, 4)]
    return pl.pallas_call(
        functools.partial(_tc_loss_kernel, N, weights),
        grid=(grid,),
        in_specs=sspec + [pl.BlockSpec((3, _BL), lambda i: (0, i)),
                          pl.BlockSpec((_BL,), lambda i: (i,))],
        out_specs=pl.BlockSpec((1, 1), lambda i: (0, 0)),
        out_shape=jax.ShapeDtypeStruct((1, 1), jnp.float32),
        scratch_shapes=[pltpu.VMEM((3, _BL), jnp.float32)],
    )(st, st, st, st, xt_t, invw)


def kernel(states_x, x_target, molecule_id):
    N = molecule_id.shape[0]
    K = states_x.shape[0] - 1
    w = [_GAMMA ** (K - 1 - k) for k in range(K)]
    tot = sum(w)
    weights = tuple(float(x / tot) for x in w)

    invw = _sc_invw(molecule_id)

    st = jnp.transpose(states_x, (0, 2, 1))
    xt_t = jnp.transpose(x_target, (1, 0))
    loss = _tc_loss(st, xt_t, invw, weights, N)
    return loss[0, 0]
